# baseline (speedup 1.0000x reference)
import functools

import jax
import jax.numpy as jnp
from jax import lax
from jax.experimental import pallas as pl
from jax.experimental.pallas import tpu as pltpu
from jax.experimental.pallas import tpu_sc as plsc

_R = 3
_CH = 2048
_QPR = 64 * 32768 // _CH
_Q = _R * _QPR
_NW = 32
_QPW = _Q // _NW
_B = 24
_NB = _QPW // _B


def _idx_body(mask_ref, idx_ref):
    q = (lax.broadcasted_iota(jnp.int32, idx_ref.shape, 0) * idx_ref.shape[1]
         + lax.broadcasted_iota(jnp.int32, idx_ref.shape, 1))
    rowq = q // _QPR
    colq = q - rowq * _QPR
    src_vec = jnp.zeros(idx_ref.shape, jnp.int32)
    for r in range(_R):
        count = 0
        src_r = 0
        for row in range(_R):
            keep = 1 - mask_ref[row]
            hit = jnp.logical_and(count == r, keep == 1)
            src_r = jnp.where(hit, row, src_r)
            count = count + keep
        src_vec = jnp.where(rowq == r, src_r, src_vec)
    idx_ref[...] = src_vec * _QPR + colq


def _compute_idx(mask_i32):
    return pl.pallas_call(
        _idx_body,
        in_specs=[pl.BlockSpec(memory_space=pltpu.SMEM)],
        out_specs=pl.BlockSpec((_Q // 128, 128), lambda: (0, 0)),
        out_shape=jax.ShapeDtypeStruct((_Q // 128, 128), jnp.int32),
    )(mask_i32)


def _make_sc_gather():
    mesh = plsc.VectorSubcoreMesh(core_axis_name="c", subcore_axis_name="s")

    @functools.partial(
        pl.kernel,
        mesh=mesh,
        out_type=jax.ShapeDtypeStruct((_Q, _CH), jnp.float32),
        scratch_types=[
            pltpu.VMEM((_QPW,), jnp.int32),
            pltpu.VMEM((_B, _CH), jnp.float32),
            pltpu.VMEM((_B, _CH), jnp.float32),
            pltpu.SemaphoreType.DMA,
            pltpu.SemaphoreType.DMA,
            pltpu.SemaphoreType.DMA,
            pltpu.SemaphoreType.DMA,
        ],
    )
    def sc_gather(x_hbm, idx_hbm, out_hbm, idx_v, buf0, buf1,
                  in_sem0, in_sem1, out_sem0, out_sem1):
        wid = lax.axis_index("s") * 2 + lax.axis_index("c")
        base = wid * _QPW
        pltpu.sync_copy(idx_hbm.at[pl.ds(base, _QPW)], idx_v)

        bufs = (buf0, buf1)
        in_sems = (in_sem0, in_sem1)
        out_sems = (out_sem0, out_sem1)
        stores = [None, None]
        for t in range(_NB):
            b = t % 2
            if stores[b] is not None:
                stores[b].wait()
            gather = pltpu.make_async_copy(
                x_hbm.at[idx_v.at[pl.ds(t * _B, _B)]], bufs[b], in_sems[b]
            )
            gather.start()
            gather.wait()
            store = pltpu.make_async_copy(
                bufs[b], out_hbm.at[pl.ds(base + t * _B, _B)], out_sems[b]
            )
            store.start()
            stores[b] = store
        for b in range(2):
            if stores[b] is not None:
                stores[b].wait()

    return sc_gather


_sc_gather = _make_sc_gather()


def kernel(x, bool_tensor):
    mask_i32 = bool_tensor.astype(jnp.int32)
    src = jnp.nonzero(mask_i32 == 0, size=_R)[0].astype(jnp.int32)
    q = lax.iota(jnp.int32, _Q)
    idx = src[q // _QPR] * _QPR + q % _QPR
    out = _sc_gather(x.reshape(_Q, _CH), idx)
    return out.reshape(_R, 64, 32768)


# speedup vs baseline: 5.2938x; 5.2938x over previous
import jax
import jax.numpy as jnp
from jax.experimental import pallas as pl
from jax.experimental.pallas import tpu as pltpu

_R = 3
_M = 64
_N = 32768
_W = 16384


def _copy_body(mask_ref, x_ref, o_ref):
    o_ref[...] = x_ref[...]


def _src_index_map(i, j, mask_ref):
    count = 0
    src = 0
    for row in range(_R):
        keep = 1 - mask_ref[row]
        hit = jnp.logical_and(count == i, keep == 1)
        src = jnp.where(hit, row, src)
        count = count + keep
    return (src, 0, j)


def kernel(x, bool_tensor):
    mask_i32 = bool_tensor.astype(jnp.int32)
    grid = (_R, _N // _W)
    out = pl.pallas_call(
        _copy_body,
        grid_spec=pltpu.PrefetchScalarGridSpec(
            num_scalar_prefetch=1,
            grid=grid,
            in_specs=[
                pl.BlockSpec((1, _M, _W), _src_index_map),
            ],
            out_specs=pl.BlockSpec((1, _M, _W), lambda i, j, m: (i, 0, j)),
        ),
        out_shape=jax.ShapeDtypeStruct((_R, _M, _N), x.dtype),
    )(mask_i32, x)
    return out


# speedup vs baseline: 5.8041x; 1.0964x over previous
import jax
import jax.numpy as jnp
from jax.experimental import pallas as pl
from jax.experimental.pallas import tpu as pltpu

_R = 3
_M = 64
_N = 32768
_W = 32768


def _copy_body(mask_ref, x_ref, o_ref):
    o_ref[...] = x_ref[...]


def _src_index_map(i, j, mask_ref):
    count = 0
    src = 0
    for row in range(_R):
        keep = 1 - mask_ref[row]
        hit = jnp.logical_and(count == i, keep == 1)
        src = jnp.where(hit, row, src)
        count = count + keep
    return (src, 0, j)


def kernel(x, bool_tensor):
    mask_i32 = bool_tensor.astype(jnp.int32)
    grid = (_R, _N // _W)
    out = pl.pallas_call(
        _copy_body,
        grid_spec=pltpu.PrefetchScalarGridSpec(
            num_scalar_prefetch=1,
            grid=grid,
            in_specs=[
                pl.BlockSpec((1, _M, _W), _src_index_map),
            ],
            out_specs=pl.BlockSpec((1, _M, _W), lambda i, j, m: (i, 0, j)),
        ),
        out_shape=jax.ShapeDtypeStruct((_R, _M, _N), x.dtype),
    )(mask_i32, x)
    return out
